# baseline (speedup 1.0000x reference)
import functools

import jax
import jax.numpy as jnp
from jax import lax
from jax.experimental import pallas as pl
from jax.experimental.pallas import tpu as pltpu
from jax.experimental.pallas import tpu_sc as plsc

N = 10000
E = 320000
D = 128
G = 64

NC = 2
NS = 16
NW = NC * NS
N_PAD = 10240
CHUNK = 128
EPW = E // NW
WCH = 16
CPW = 80
NWIN = CPW // WCH
EPW_PAD = CPW * CHUNK
RPT = N_PAD // NS
ROW_BLK = 1024
N_BLKS = N_PAD // ROW_BLK

@functools.cache
def _sc_mesh():
    return plsc.VectorSubcoreMesh(core_axis_name="c", subcore_axis_name="s",
                                  num_cores=NC, num_subcores=NS)


def _deg_body(dst_hbm, out_hbm, dst_v, ones_v, buf, deg_sh):
    c = lax.axis_index("c")
    s = lax.axis_index("s")
    wid = s * NC + c

    @pl.loop(0, CHUNK)
    def _(j):
        @pl.loop(0, D // 16)
        def _(q):
            ones_v[j, pl.ds(q * 16, 16)] = jnp.ones((16,), jnp.float32)
            buf[j, pl.ds(q * 16, 16)] = jnp.zeros((16,), jnp.float32)

    @pl.loop(0, RPT // CHUNK)
    def _(k):
        pltpu.sync_copy(buf, deg_sh.at[pl.ds(s * RPT + k * CHUNK, CHUNK)])

    pltpu.sync_copy(dst_hbm.at[pl.ds(wid * CPW, CPW)], dst_v)
    plsc.subcore_barrier()

    @pl.loop(0, CPW)
    def _(i):
        pltpu.sync_copy(ones_v, deg_sh.at[dst_v.at[i]], add=True)

    plsc.subcore_barrier()

    @pl.loop(0, RPT // CHUNK)
    def _(k):
        off = s * RPT + k * CHUNK
        pltpu.sync_copy(deg_sh.at[pl.ds(off, CHUNK)], buf)
        pltpu.sync_copy(buf, out_hbm.at[c, pl.ds(off, CHUNK)])


@functools.cache
def _deg_kernel():
    return pl.kernel(
        _deg_body,
        out_type=jax.ShapeDtypeStruct((NC, N_PAD, D), jnp.float32),
        mesh=_sc_mesh(),
        scratch_types=[
            pltpu.VMEM((CPW, CHUNK), jnp.int32),
            pltpu.VMEM((CHUNK, D), jnp.float32),
            pltpu.VMEM((CHUNK, D), jnp.float32),
            pltpu.VMEM_SHARED((N_PAD, D), jnp.float32),
        ],
    )


PH = 2
PCH = CPW // PH


def _agg_body(y_hbm, src_hbm, dst_hbm, out_hbm, src_v, dst_v, bufa,
              acc_sh, sga):
    c = lax.axis_index("c")
    s = lax.axis_index("s")
    wid = s * NC + c

    @pl.loop(0, RPT // CHUNK)
    def _(k):
        off = s * RPT + k * CHUNK
        pltpu.sync_copy(y_hbm.at[pl.ds(off, CHUNK)], bufa)
        pltpu.sync_copy(bufa, acc_sh.at[pl.ds(off, CHUNK)])

    plsc.subcore_barrier()

    pltpu.sync_copy(src_hbm.at[pl.ds(wid * CPW, CPW)], src_v)
    pltpu.sync_copy(dst_hbm.at[pl.ds(wid * CPW, CPW)], dst_v)

    @pl.loop(0, CPW)
    def _(i):
        pltpu.async_copy(y_hbm.at[src_v.at[i]], bufa, sga).wait()
        pltpu.sync_copy(bufa, acc_sh.at[dst_v.at[i]], add=True)

    plsc.subcore_barrier()

    @pl.loop(0, RPT // CHUNK)
    def _(k):
        off = s * RPT + k * CHUNK
        pltpu.sync_copy(acc_sh.at[pl.ds(off, CHUNK)], bufa)
        pltpu.sync_copy(bufa, out_hbm.at[c, pl.ds(off, CHUNK)])


@functools.cache
def _agg_kernel():
    return pl.kernel(
        _agg_body,
        out_type=jax.ShapeDtypeStruct((NC, N_PAD, D), jnp.float32),
        mesh=_sc_mesh(),
        scratch_types=[
            pltpu.VMEM((CPW, CHUNK), jnp.int32),
            pltpu.VMEM((CPW, CHUNK), jnp.int32),
            pltpu.VMEM((CHUNK, D), jnp.float32),
            pltpu.VMEM_SHARED((N_PAD, D), jnp.float32),
            pltpu.SemaphoreType.DMA,
        ],
    )


def _t1_body(x_ref, w_ref, deg_ref, y_ref, dinv_ref):
    x = x_ref[...]
    nrm = jnp.sqrt(jnp.sum(x * x, axis=1, keepdims=True))
    nrm = jnp.where(nrm == 0, 1e-8, nrm)
    h = x / nrm
    xw = jnp.dot(h, w_ref[...], preferred_element_type=jnp.float32)
    d = deg_ref[...]
    deg = d[0, :, 0:1] + d[1, :, 0:1] + 1.0
    dinv = lax.rsqrt(deg)
    y_ref[...] = xw * dinv
    dinv_ref[...] = dinv


def _t2_body(acc_ref, y_ref, dinv_ref, w_ref, b_ref, out_ref):
    a = acc_ref[0] + acc_ref[1]
    dinv = dinv_ref[...]
    pre = (a - y_ref[...]) * dinv + b_ref[...][None, :]
    g = jax.nn.gelu(pre)
    out_ref[...] = jnp.dot(g, w_ref[...], preferred_element_type=jnp.float32) * dinv


def _t3_body(acc_ref, y_ref, dinv_ref, b_ref, batch_ref, wfc_ref, bfc_ref,
             out_ref, pooled):
    i = pl.program_id(0)

    @pl.when(i == 0)
    def _():
        pooled[...] = jnp.zeros((G, D), jnp.float32)

    a = acc_ref[0] + acc_ref[1]
    pre = (a - y_ref[...]) * dinv_ref[...] + b_ref[...][None, :]
    g = jax.nn.gelu(pre)
    oh = (batch_ref[...] == lax.broadcasted_iota(jnp.int32, (ROW_BLK, G), 1))
    oh = oh.astype(jnp.float32)
    pooled[...] += lax.dot_general(oh, g, (((0,), (0,)), ((), ())),
                                   preferred_element_type=jnp.float32)

    @pl.when(i == N_BLKS - 1)
    def _():
        p = pooled[...]
        o = jnp.sum(p * wfc_ref[...][None, :], axis=1) + bfc_ref[...]
        out_ref[...] = jax.nn.gelu(o)[:, None]


def kernel(x, edge_index, batch, W1, b1, W2, b2, Wfc, bfc):
    f32 = jnp.float32
    src = edge_index[0].reshape(NW, EPW)
    dst = edge_index[1].reshape(NW, EPW)
    srcp = jnp.pad(src, ((0, 0), (0, EPW_PAD - EPW))).reshape(NW * CPW, CHUNK)
    dstp = jnp.pad(dst, ((0, 0), (0, EPW_PAD - EPW)),
                   constant_values=N).reshape(NW * CPW, CHUNK)
    xp = jnp.pad(x, ((0, N_PAD - N), (0, 0)))
    batchp = jnp.pad(batch, (0, N_PAD - N), constant_values=G).reshape(N_PAD, 1)

    deg = _deg_kernel()(dstp)

    y1, dinv = pl.pallas_call(
        _t1_body,
        grid=(N_BLKS,),
        in_specs=[
            pl.BlockSpec((ROW_BLK, D), lambda i: (i, 0)),
            pl.BlockSpec((D, D), lambda i: (0, 0)),
            pl.BlockSpec((NC, ROW_BLK, D), lambda i: (0, i, 0)),
        ],
        out_specs=[
            pl.BlockSpec((ROW_BLK, D), lambda i: (i, 0)),
            pl.BlockSpec((ROW_BLK, 1), lambda i: (i, 0)),
        ],
        out_shape=[
            jax.ShapeDtypeStruct((N_PAD, D), f32),
            jax.ShapeDtypeStruct((N_PAD, 1), f32),
        ],
    )(xp, W1, deg)

    acc1 = _agg_kernel()(y1, srcp, dstp)

    y2 = pl.pallas_call(
        _t2_body,
        grid=(N_BLKS,),
        in_specs=[
            pl.BlockSpec((NC, ROW_BLK, D), lambda i: (0, i, 0)),
            pl.BlockSpec((ROW_BLK, D), lambda i: (i, 0)),
            pl.BlockSpec((ROW_BLK, 1), lambda i: (i, 0)),
            pl.BlockSpec((D, D), lambda i: (0, 0)),
            pl.BlockSpec((D,), lambda i: (0,)),
        ],
        out_specs=pl.BlockSpec((ROW_BLK, D), lambda i: (i, 0)),
        out_shape=jax.ShapeDtypeStruct((N_PAD, D), f32),
    )(acc1, y1, dinv, W2, b1)

    acc2 = _agg_kernel()(y2, srcp, dstp)

    out = pl.pallas_call(
        _t3_body,
        grid=(N_BLKS,),
        in_specs=[
            pl.BlockSpec((NC, ROW_BLK, D), lambda i: (0, i, 0)),
            pl.BlockSpec((ROW_BLK, D), lambda i: (i, 0)),
            pl.BlockSpec((ROW_BLK, 1), lambda i: (i, 0)),
            pl.BlockSpec((D,), lambda i: (0,)),
            pl.BlockSpec((ROW_BLK, 1), lambda i: (i, 0)),
            pl.BlockSpec((D,), lambda i: (0,)),
            pl.BlockSpec((1,), lambda i: (0,)),
        ],
        out_specs=pl.BlockSpec((G, 1), lambda i: (0, 0)),
        out_shape=jax.ShapeDtypeStruct((G, 1), f32),
        scratch_shapes=[pltpu.VMEM((G, D), f32)],
    )(acc2, y2, dinv, b2, batchp, Wfc.reshape(D), bfc)

    return out[:, 0]


# speedup vs baseline: 1.4144x; 1.4144x over previous
import functools

import jax
import jax.numpy as jnp
from jax import lax
from jax.experimental import pallas as pl
from jax.experimental.pallas import tpu as pltpu
from jax.experimental.pallas import tpu_sc as plsc

N = 10000
E = 320000
D = 128
G = 64

NC = 2
NS = 16
NW = NC * NS
N_PAD = 10240
CHUNK = 128
EPW = E // NW
CPW = -(-EPW // CHUNK)
EPW_PAD = CPW * CHUNK
RPT = N_PAD // NS
ROW_BLK = 1024
N_BLKS = N_PAD // ROW_BLK

@functools.cache
def _sc_mesh():
    return plsc.VectorSubcoreMesh(core_axis_name="c", subcore_axis_name="s",
                                  num_cores=NC, num_subcores=NS)


def _deg_body(dst_hbm, out_hbm, dst_v, ones_v, buf, deg_sh):
    c = lax.axis_index("c")
    s = lax.axis_index("s")
    wid = s * NC + c

    @pl.loop(0, CHUNK)
    def _(j):
        @pl.loop(0, D // 16)
        def _(q):
            ones_v[j, pl.ds(q * 16, 16)] = jnp.ones((16,), jnp.float32)
            buf[j, pl.ds(q * 16, 16)] = jnp.zeros((16,), jnp.float32)

    @pl.loop(0, RPT // CHUNK)
    def _(k):
        pltpu.sync_copy(buf, deg_sh.at[pl.ds(s * RPT + k * CHUNK, CHUNK)])

    pltpu.sync_copy(dst_hbm.at[wid], dst_v)
    plsc.subcore_barrier()

    @pl.loop(0, CPW)
    def _(i):
        pltpu.sync_copy(ones_v, deg_sh.at[dst_v.at[i]], add=True)

    plsc.subcore_barrier()

    @pl.loop(0, RPT // CHUNK)
    def _(k):
        off = s * RPT + k * CHUNK
        pltpu.sync_copy(deg_sh.at[pl.ds(off, CHUNK)], buf)
        pltpu.sync_copy(buf, out_hbm.at[c, pl.ds(off, CHUNK)])


@functools.cache
def _deg_kernel():
    return pl.kernel(
        _deg_body,
        out_type=jax.ShapeDtypeStruct((NC, N_PAD, D), jnp.float32),
        mesh=_sc_mesh(),
        scratch_types=[
            pltpu.VMEM((CPW, CHUNK), jnp.int32),
            pltpu.VMEM((CHUNK, D), jnp.float32),
            pltpu.VMEM((CHUNK, D), jnp.float32),
            pltpu.VMEM_SHARED((N_PAD, D), jnp.float32),
        ],
    )


def _agg_body(y_hbm, src_hbm, dst_hbm, out_hbm, src_v, dst_v, bufa,
              acc_sh, sga):
    c = lax.axis_index("c")
    s = lax.axis_index("s")
    wid = s * NC + c

    pltpu.sync_copy(src_hbm.at[wid], src_v)
    pltpu.sync_copy(dst_hbm.at[wid], dst_v)

    @pl.loop(0, RPT // CHUNK)
    def _(k):
        off = s * RPT + k * CHUNK
        pltpu.sync_copy(y_hbm.at[pl.ds(off, CHUNK)], bufa)
        pltpu.sync_copy(bufa, acc_sh.at[pl.ds(off, CHUNK)])

    plsc.subcore_barrier()

    @pl.loop(0, CPW)
    def _(i):
        pltpu.async_copy(y_hbm.at[src_v.at[i]], bufa, sga).wait()
        pltpu.sync_copy(bufa, acc_sh.at[dst_v.at[i]], add=True)

    plsc.subcore_barrier()

    @pl.loop(0, RPT // CHUNK)
    def _(k):
        off = s * RPT + k * CHUNK
        pltpu.sync_copy(acc_sh.at[pl.ds(off, CHUNK)], bufa)
        pltpu.sync_copy(bufa, out_hbm.at[c, pl.ds(off, CHUNK)])


@functools.cache
def _agg_kernel():
    return pl.kernel(
        _agg_body,
        out_type=jax.ShapeDtypeStruct((NC, N_PAD, D), jnp.float32),
        mesh=_sc_mesh(),
        scratch_types=[
            pltpu.VMEM((CPW, CHUNK), jnp.int32),
            pltpu.VMEM((CPW, CHUNK), jnp.int32),
            pltpu.VMEM((CHUNK, D), jnp.float32),
            pltpu.VMEM_SHARED((N_PAD, D), jnp.float32),
            pltpu.SemaphoreType.DMA,
        ],
    )


def _t1_body(x_ref, w_ref, deg_ref, y_ref, dinv_ref):
    x = x_ref[...]
    nrm = jnp.sqrt(jnp.sum(x * x, axis=1, keepdims=True))
    nrm = jnp.where(nrm == 0, 1e-8, nrm)
    h = x / nrm
    xw = jnp.dot(h, w_ref[...], preferred_element_type=jnp.float32)
    d = deg_ref[...]
    deg = d[0, :, 0:1] + d[1, :, 0:1] + 1.0
    dinv = lax.rsqrt(deg)
    y_ref[...] = xw * dinv
    dinv_ref[...] = dinv


def _t2_body(acc_ref, y_ref, dinv_ref, w_ref, b_ref, out_ref):
    a = acc_ref[0] + acc_ref[1]
    dinv = dinv_ref[...]
    pre = (a - y_ref[...]) * dinv + b_ref[...][None, :]
    g = jax.nn.gelu(pre)
    out_ref[...] = jnp.dot(g, w_ref[...], preferred_element_type=jnp.float32) * dinv


def _t3_body(acc_ref, y_ref, dinv_ref, b_ref, batch_ref, wfc_ref, bfc_ref,
             out_ref, pooled):
    i = pl.program_id(0)

    @pl.when(i == 0)
    def _():
        pooled[...] = jnp.zeros((G, D), jnp.float32)

    a = acc_ref[0] + acc_ref[1]
    pre = (a - y_ref[...]) * dinv_ref[...] + b_ref[...][None, :]
    g = jax.nn.gelu(pre)
    oh = (batch_ref[...] == lax.broadcasted_iota(jnp.int32, (ROW_BLK, G), 1))
    oh = oh.astype(jnp.float32)
    pooled[...] += lax.dot_general(oh, g, (((0,), (0,)), ((), ())),
                                   preferred_element_type=jnp.float32)

    @pl.when(i == N_BLKS - 1)
    def _():
        p = pooled[...]
        o = jnp.sum(p * wfc_ref[...][None, :], axis=1) + bfc_ref[...]
        out_ref[...] = jax.nn.gelu(o)[:, None]


def kernel(x, edge_index, batch, W1, b1, W2, b2, Wfc, bfc):
    f32 = jnp.float32
    src = edge_index[0].reshape(NW, EPW)
    dst = edge_index[1].reshape(NW, EPW)
    srcp = jnp.pad(src, ((0, 0), (0, EPW_PAD - EPW))).reshape(NW, CPW, CHUNK)
    dstp = jnp.pad(dst, ((0, 0), (0, EPW_PAD - EPW)),
                   constant_values=N).reshape(NW, CPW, CHUNK)
    xp = jnp.pad(x, ((0, N_PAD - N), (0, 0)))
    batchp = jnp.pad(batch, (0, N_PAD - N), constant_values=G).reshape(N_PAD, 1)

    deg = _deg_kernel()(dstp)

    y1, dinv = pl.pallas_call(
        _t1_body,
        grid=(N_BLKS,),
        in_specs=[
            pl.BlockSpec((ROW_BLK, D), lambda i: (i, 0)),
            pl.BlockSpec((D, D), lambda i: (0, 0)),
            pl.BlockSpec((NC, ROW_BLK, D), lambda i: (0, i, 0)),
        ],
        out_specs=[
            pl.BlockSpec((ROW_BLK, D), lambda i: (i, 0)),
            pl.BlockSpec((ROW_BLK, 1), lambda i: (i, 0)),
        ],
        out_shape=[
            jax.ShapeDtypeStruct((N_PAD, D), f32),
            jax.ShapeDtypeStruct((N_PAD, 1), f32),
        ],
    )(xp, W1, deg)

    acc1 = _agg_kernel()(y1, srcp, dstp)

    y2 = pl.pallas_call(
        _t2_body,
        grid=(N_BLKS,),
        in_specs=[
            pl.BlockSpec((NC, ROW_BLK, D), lambda i: (0, i, 0)),
            pl.BlockSpec((ROW_BLK, D), lambda i: (i, 0)),
            pl.BlockSpec((ROW_BLK, 1), lambda i: (i, 0)),
            pl.BlockSpec((D, D), lambda i: (0, 0)),
            pl.BlockSpec((D,), lambda i: (0,)),
        ],
        out_specs=pl.BlockSpec((ROW_BLK, D), lambda i: (i, 0)),
        out_shape=jax.ShapeDtypeStruct((N_PAD, D), f32),
    )(acc1, y1, dinv, W2, b1)

    acc2 = _agg_kernel()(y2, srcp, dstp)

    out = pl.pallas_call(
        _t3_body,
        grid=(N_BLKS,),
        in_specs=[
            pl.BlockSpec((NC, ROW_BLK, D), lambda i: (0, i, 0)),
            pl.BlockSpec((ROW_BLK, D), lambda i: (i, 0)),
            pl.BlockSpec((ROW_BLK, 1), lambda i: (i, 0)),
            pl.BlockSpec((D,), lambda i: (0,)),
            pl.BlockSpec((ROW_BLK, 1), lambda i: (i, 0)),
            pl.BlockSpec((D,), lambda i: (0,)),
            pl.BlockSpec((1,), lambda i: (0,)),
        ],
        out_specs=pl.BlockSpec((G, 1), lambda i: (0, 0)),
        out_shape=jax.ShapeDtypeStruct((G, 1), f32),
        scratch_shapes=[pltpu.VMEM((G, D), f32)],
    )(acc2, y2, dinv, b2, batchp, Wfc.reshape(D), bfc)

    return out[:, 0]
